# baseline (speedup 1.0000x reference)
import functools

import jax
import jax.numpy as jnp
from jax import lax
from jax.experimental import pallas as pl
from jax.experimental.pallas import tpu as pltpu
from jax.experimental.pallas import tpu_sc as plsc

NUM_GRAPHS = 128
HIDDEN = 256
BLOCK = 1024

NC = 2
NS = 16
NW = NC * NS


def _make_sc_sum(npad):
    rows_per_w = npad // NW
    tile = 224
    while rows_per_w % tile:
        tile //= 2
    n_tiles = rows_per_w // tile
    mesh = plsc.VectorSubcoreMesh(core_axis_name="c", subcore_axis_name="s")

    nst = NUM_GRAPHS + 32
    ntl = NW * n_tiles + 16

    def _sread(ref, i):
        return ref[pl.ds(i, 16)][0]

    @functools.partial(
        pl.kernel, mesh=mesh,
        out_type=jax.ShapeDtypeStruct((NW, NUM_GRAPHS, HIDDEN), jnp.float32),
        scratch_types=[
            pltpu.VMEM((NUM_GRAPHS, HIDDEN), jnp.float32),
            pltpu.VMEM((tile, HIDDEN), jnp.float32),
            pltpu.VMEM((nst,), jnp.int32),
            pltpu.VMEM((ntl,), jnp.int32),
            pltpu.VMEM((ntl,), jnp.int32),
        ],
    )
    def sc_sum(x_hbm, st_hbm, tf_hbm, tl_hbm, z_hbm, o_hbm,
               acc_v, rows_v, st_sm, tf_sm, tl_sm):
        cid = lax.axis_index("c")
        sid = lax.axis_index("s")
        wid = sid * NC + cid

        pltpu.sync_copy(z_hbm, acc_v)
        pltpu.sync_copy(st_hbm, st_sm)
        pltpu.sync_copy(tf_hbm, tf_sm)
        pltpu.sync_copy(tl_hbm, tl_sm)

        @pl.loop(0, n_tiles)
        def _(t):
            base = wid * rows_per_w + t * tile
            pltpu.sync_copy(x_hbm.at[pl.ds(base, tile)], rows_v)
            tfirst = _sread(tf_sm, wid * n_tiles + t)
            tlast = _sread(tl_sm, wid * n_tiles + t)

            @pl.loop(tfirst, tlast + 1)
            def _(s):
                r0 = jnp.maximum(_sread(st_sm, s) - base, 0)
                r1 = jnp.maximum(
                    jnp.minimum(_sread(st_sm, s + 1) - base, tile), r0)
                for c in range(HIDDEN // 16):
                    sl = pl.ds(c * 16, 16)

                    @pl.loop(r0, r1)
                    def _(r):
                        acc_v[s, sl] += rows_v[r, sl]

        pltpu.sync_copy(acc_v, o_hbm.at[wid])

    return sc_sum, tile


def _max_kernel(firsts, lasts, x_ref, seg_ref, o_ref):
    i = pl.program_id(0)

    @pl.when(i == 0)
    def _():
        o_ref[...] = jnp.full_like(o_ref, -jnp.inf)

    first = firsts[i]
    last = lasts[i]

    def body(s, carry):
        acc_mx = jnp.full((8, HIDDEN), -jnp.inf, dtype=jnp.float32)
        for k in range(BLOCK // 64):
            xk = x_ref[k * 64:(k + 1) * 64, :]
            mk = seg_ref[k * 64:(k + 1) * 64, :] == s
            xm = jnp.where(mk, xk, -jnp.inf).reshape(8, 8, HIDDEN)
            acc_mx = jnp.maximum(acc_mx, jnp.max(xm, axis=0))
        bmax = jnp.max(acc_mx, axis=0, keepdims=True)
        o_ref[pl.ds(s, 1), :] = jnp.maximum(o_ref[pl.ds(s, 1), :], bmax)
        return carry

    jax.lax.fori_loop(first, last + 1, body, 0)


def _combine_kernel(scp_ref, mx_ref, sv_ref, wt_ref, b_ref, o_ref):
    sums = jnp.sum(scp_ref[...], axis=0)
    sv = sv_ref[...]
    counts = sv[1:NUM_GRAPHS + 1, :] - sv[:NUM_GRAPHS, :]
    mean = sums / jnp.maximum(counts, 1.0)
    comb = jnp.concatenate([mx_ref[...], mean], axis=1)
    o_ref[...] = jax.lax.dot_general(
        comb, wt_ref[...], (((1,), (0,)), ((), ())),
        preferred_element_type=jnp.float32) + b_ref[...]


@jax.jit
def kernel(x, batch, W, b):
    n, h = x.shape
    batch = batch.astype(jnp.int32)
    npad = ((n + BLOCK - 1) // BLOCK) * BLOCK
    nb = npad // BLOCK
    xp = jnp.pad(x, ((0, npad - n), (0, 0)))
    segp = jnp.pad(batch, (0, npad - n), constant_values=NUM_GRAPHS)
    segc = jnp.minimum(segp, NUM_GRAPHS - 1)
    firsts = segp[::BLOCK]
    lasts = jnp.minimum(segp[BLOCK - 1::BLOCK], NUM_GRAPHS - 1)
    seg2d = segp.reshape(npad, 1)
    starts = jnp.searchsorted(batch, jnp.arange(NUM_GRAPHS + 1,
                                                dtype=jnp.int32)
                              ).astype(jnp.int32)
    sv = jnp.pad(starts.astype(jnp.float32),
                 (0, 7)).reshape(NUM_GRAPHS + 8, 1)
    wt = W.T
    b2 = b.reshape(1, h)
    zeros = jnp.zeros((NUM_GRAPHS, h), jnp.float32)

    sc_sum, tile = _make_sc_sum(npad)
    starts_i = jnp.pad(starts, (0, 31))
    tf = jnp.pad(segc[::tile], (0, 16))
    tl = jnp.pad(segc[tile - 1::tile], (0, 16))
    sc_parts = sc_sum(xp, starts_i, tf, tl, zeros)

    mx = pl.pallas_call(
        _max_kernel,
        grid_spec=pltpu.PrefetchScalarGridSpec(
            num_scalar_prefetch=2,
            grid=(nb,),
            in_specs=[
                pl.BlockSpec((BLOCK, h), lambda i, *_: (i, 0)),
                pl.BlockSpec((BLOCK, 1), lambda i, *_: (i, 0)),
            ],
            out_specs=pl.BlockSpec((NUM_GRAPHS, h), lambda i, *_: (0, 0)),
        ),
        out_shape=jax.ShapeDtypeStruct((NUM_GRAPHS, h), jnp.float32),
    )(firsts, lasts, xp, seg2d)

    out = pl.pallas_call(
        _combine_kernel,
        in_specs=[
            pl.BlockSpec((NW, NUM_GRAPHS, h), lambda: (0, 0, 0)),
            pl.BlockSpec((NUM_GRAPHS, h), lambda: (0, 0)),
            pl.BlockSpec((NUM_GRAPHS + 8, 1), lambda: (0, 0)),
            pl.BlockSpec((2 * h, h), lambda: (0, 0)),
            pl.BlockSpec((1, h), lambda: (0, 0)),
        ],
        out_specs=pl.BlockSpec((NUM_GRAPHS, h), lambda: (0, 0)),
        out_shape=jax.ShapeDtypeStruct((NUM_GRAPHS, h), jnp.float32),
    )(sc_parts, mx, sv, wt, b2)
    return out


# speedup vs baseline: 1.6822x; 1.6822x over previous
import functools

import jax
import jax.numpy as jnp
from jax import lax
from jax.experimental import pallas as pl
from jax.experimental.pallas import tpu as pltpu
from jax.experimental.pallas import tpu_sc as plsc

NUM_GRAPHS = 128
HIDDEN = 256
NLANE = 16
NCHUNK = HIDDEN // NLANE
HALF = NCHUNK // 2

NC = 2
NS = 16
NW = NC * NS
NEG_INF = float("-inf")


def _make_sc_pool(npad):
    rows_per_w = npad // NW
    tile = 112
    while rows_per_w % tile:
        tile //= 2
    n_tiles = rows_per_w // tile
    nst = NUM_GRAPHS + 32
    ntl = NW * n_tiles + 16
    mesh = plsc.VectorSubcoreMesh(core_axis_name="c", subcore_axis_name="s")

    def _sread(ref, i):
        return ref[pl.ds(i, 16)][0]

    @functools.partial(
        pl.kernel, mesh=mesh,
        out_type=jax.ShapeDtypeStruct((NW, 2, NUM_GRAPHS, HIDDEN),
                                      jnp.float32),
        scratch_types=[
            pltpu.VMEM((NUM_GRAPHS, HIDDEN), jnp.float32),
            pltpu.VMEM((NUM_GRAPHS, HIDDEN), jnp.float32),
            pltpu.VMEM((tile, HIDDEN), jnp.float32),
            pltpu.VMEM((nst,), jnp.int32),
            pltpu.VMEM((ntl,), jnp.int32),
            pltpu.VMEM((ntl,), jnp.int32),
        ],
    )
    def sc_pool(x_hbm, st_hbm, tf_hbm, tl_hbm, z_hbm, ninf_hbm, o_hbm,
                accs_v, accm_v, rows_v, st_sm, tf_sm, tl_sm):
        cid = lax.axis_index("c")
        sid = lax.axis_index("s")
        wid = sid * NC + cid

        pltpu.sync_copy(z_hbm, accs_v)
        pltpu.sync_copy(ninf_hbm, accm_v)
        pltpu.sync_copy(st_hbm, st_sm)
        pltpu.sync_copy(tf_hbm, tf_sm)
        pltpu.sync_copy(tl_hbm, tl_sm)

        @pl.loop(0, n_tiles)
        def _(t):
            base = wid * rows_per_w + t * tile
            pltpu.sync_copy(x_hbm.at[pl.ds(base, tile)], rows_v)
            tfirst = _sread(tf_sm, wid * n_tiles + t)
            tlast = _sread(tl_sm, wid * n_tiles + t)

            @pl.loop(tfirst, tlast + 1)
            def _(s):
                r0 = jnp.maximum(_sread(st_sm, s) - base, 0)
                r1 = jnp.maximum(
                    jnp.minimum(_sread(st_sm, s + 1) - base, tile), r0)
                for half in range(2):
                    c0 = half * HALF

                    def body(r, carry):
                        out = []
                        for c in range(HALF):
                            v = rows_v[r, pl.ds((c0 + c) * NLANE, NLANE)]
                            out.append(carry[c] + v)
                            out.append(jnp.maximum(carry[HALF + c], v))
                        return tuple(out[::2]) + tuple(out[1::2])

                    init = tuple(
                        jnp.zeros((NLANE,), jnp.float32)
                        for _ in range(HALF)) + tuple(
                        jnp.full((NLANE,), NEG_INF, jnp.float32)
                        for _ in range(HALF))
                    res = lax.fori_loop(r0, r1, body, init)
                    for c in range(HALF):
                        sl = pl.ds((c0 + c) * NLANE, NLANE)
                        accs_v[s, sl] += res[c]
                        accm_v[s, sl] = jnp.maximum(accm_v[s, sl],
                                                    res[HALF + c])

        pltpu.sync_copy(accs_v, o_hbm.at[wid, 0])
        pltpu.sync_copy(accm_v, o_hbm.at[wid, 1])

    return sc_pool, tile


def _combine_kernel(p_ref, sv_ref, wt_ref, b_ref, o_ref, sm_ref, mx_ref):
    i = pl.program_id(0)

    @pl.when(i == 0)
    def _():
        sm_ref[...] = jnp.zeros_like(sm_ref)
        mx_ref[...] = jnp.full_like(mx_ref, NEG_INF)

    sm_ref[...] += p_ref[0, 0]
    mx_ref[...] = jnp.maximum(mx_ref[...], p_ref[0, 1])

    @pl.when(i == NW - 1)
    def _():
        sv = sv_ref[...]
        counts = sv[1:NUM_GRAPHS + 1, :] - sv[:NUM_GRAPHS, :]
        mean = sm_ref[...] / jnp.maximum(counts, 1.0)
        comb = jnp.concatenate([mx_ref[...], mean], axis=1)
        o_ref[...] = jax.lax.dot_general(
            comb, wt_ref[...], (((1,), (0,)), ((), ())),
            preferred_element_type=jnp.float32) + b_ref[...]


@jax.jit
def kernel(x, batch, W, b):
    n, h = x.shape
    batch = batch.astype(jnp.int32)
    npad = ((n + 8 * NW - 1) // (8 * NW)) * 8 * NW
    xp = jnp.pad(x, ((0, npad - n), (0, 0)))
    segc = jnp.minimum(
        jnp.pad(batch, (0, npad - n), constant_values=NUM_GRAPHS),
        NUM_GRAPHS - 1)
    starts = jnp.searchsorted(batch, jnp.arange(NUM_GRAPHS + 1,
                                                dtype=jnp.int32)
                              ).astype(jnp.int32)
    sv = jnp.pad(starts.astype(jnp.float32),
                 (0, 7)).reshape(NUM_GRAPHS + 8, 1)
    wt = W.T
    b2 = b.reshape(1, h)
    zeros = jnp.zeros((NUM_GRAPHS, h), jnp.float32)
    ninf = jnp.full((NUM_GRAPHS, h), NEG_INF, jnp.float32)

    sc_pool, tile = _make_sc_pool(npad)
    starts_i = jnp.pad(starts, (0, 31))
    tf = jnp.pad(segc[::tile], (0, 16))
    tl = jnp.pad(segc[tile - 1::tile], (0, 16))
    parts = sc_pool(xp, starts_i, tf, tl, zeros, ninf)

    out = pl.pallas_call(
        _combine_kernel,
        grid=(NW,),
        in_specs=[
            pl.BlockSpec((1, 2, NUM_GRAPHS, h), lambda i: (i, 0, 0, 0)),
            pl.BlockSpec((NUM_GRAPHS + 8, 1), lambda i: (0, 0)),
            pl.BlockSpec((2 * h, h), lambda i: (0, 0)),
            pl.BlockSpec((1, h), lambda i: (0, 0)),
        ],
        out_specs=pl.BlockSpec((NUM_GRAPHS, h), lambda i: (0, 0)),
        scratch_shapes=[
            pltpu.VMEM((NUM_GRAPHS, h), jnp.float32),
            pltpu.VMEM((NUM_GRAPHS, h), jnp.float32),
        ],
        out_shape=jax.ShapeDtypeStruct((NUM_GRAPHS, h), jnp.float32),
    )(parts, sv, wt, b2)
    return out
